# baseline (device time: 17661 ns/iter reference)
import os

import jax
import jax.numpy as jnp
from jax import lax
from jax.experimental import pallas as pl
from jax.experimental.pallas import tpu as pltpu

N_DEV = 4
EPS = 1e-5
N_STEP = 4
_NO_COMM = os.environ.get("KERNEL_NO_COMM", "0") == "1"


def kernel(x, gamma, beta):
    m, n_shard = x.shape
    n_global = n_shard * N_DEV
    mb = m // N_STEP
    prs, pc = mb // 128, 128

    gamma2 = gamma.reshape(1, n_shard)
    beta2 = beta.reshape(1, n_shard)

    def body(xs_ref, xn_ref, g_ref, b_ref, out_ref,
             comm_ref, send_sems, recv_sems):
        i = pl.program_id(0)
        my = lax.axis_index("i")
        if not _NO_COMM:
            barrier_sem = pltpu.get_barrier_semaphore()

            @pl.when(i == 0)
            def _():
                for k in range(1, N_DEV):
                    pl.semaphore_signal(
                        barrier_sem, inc=1,
                        device_id=(lax.rem(my + k, N_DEV),),
                        device_id_type=pl.DeviceIdType.MESH,
                    )

        row = lax.broadcasted_iota(jnp.int32, (mb, pc), 0)
        lane = lax.broadcasted_iota(jnp.int32, (mb, pc), 1)
        mask = (lane == row % pc).astype(jnp.float32)
        sel = (
            lax.broadcasted_iota(jnp.int32, (mb, prs), 1)
            == lax.broadcasted_iota(jnp.int32, (mb, prs), 0) // pc
        ).astype(jnp.float32)

        @pl.when(i < N_STEP)
        def _():
            xf = xs_ref[:, :]
            s1 = jnp.sum(xf, axis=1, keepdims=True)
            s2 = jnp.sum(xf * xf, axis=1, keepdims=True)
            comm_ref[i, 0, 0] = lax.dot_general(
                sel, s1 * mask, (((0,), (0,)), ((), ())),
                preferred_element_type=jnp.float32,
            )
            comm_ref[i, 0, 1] = lax.dot_general(
                sel, s2 * mask, (((0,), (0,)), ((), ())),
                preferred_element_type=jnp.float32,
            )

            if not _NO_COMM:
                @pl.when(i == 0)
                def _():
                    pl.semaphore_wait(barrier_sem, N_DEV - 1)

                for k in range(1, N_DEV):
                    pltpu.make_async_remote_copy(
                        src_ref=comm_ref.at[i, 0],
                        dst_ref=comm_ref.at[i, k],
                        send_sem=send_sems.at[i, k - 1],
                        recv_sem=recv_sems.at[i, k - 1],
                        device_id=(lax.rem(my + k, N_DEV),),
                        device_id_type=pl.DeviceIdType.MESH,
                    ).start()

        @pl.when(i >= 1)
        def _():
            j = i - 1
            if not _NO_COMM:
                for k in range(1, N_DEV):
                    pltpu.make_async_remote_copy(
                        src_ref=comm_ref.at[j, 0],
                        dst_ref=comm_ref.at[j, k],
                        send_sem=send_sems.at[j, k - 1],
                        recv_sem=recv_sems.at[j, k - 1],
                        device_id=(lax.rem(my + k, N_DEV),),
                        device_id_type=pl.DeviceIdType.MESH,
                    ).wait_recv()
                tot1 = (comm_ref[j, 0, 0] + comm_ref[j, 1, 0]
                        + comm_ref[j, 2, 0] + comm_ref[j, 3, 0])
                tot2 = (comm_ref[j, 0, 1] + comm_ref[j, 1, 1]
                        + comm_ref[j, 2, 1] + comm_ref[j, 3, 1])
            else:
                tot1 = comm_ref[j, 0, 0] * 4.0
                tot2 = comm_ref[j, 0, 1] * 4.0

            def unpack(t):
                u = lax.dot_general(
                    sel, t, (((1,), (0,)), ((), ())),
                    preferred_element_type=jnp.float32,
                )
                return jnp.sum(u * mask, axis=1, keepdims=True)

            inv_n = 1.0 / n_global
            mean = unpack(tot1) * inv_n
            var = unpack(tot2) * inv_n - mean * mean
            rstd = lax.rsqrt(var + EPS)
            xb = xn_ref[:, :].astype(jnp.bfloat16)
            out_ref[:, :] = (
                (xb - mean.astype(jnp.bfloat16)) * rstd.astype(jnp.bfloat16)
                * g_ref[:, :].astype(jnp.bfloat16)
                + b_ref[:, :].astype(jnp.bfloat16)
            )

            if not _NO_COMM:
                for k in range(1, N_DEV):
                    pltpu.make_async_remote_copy(
                        src_ref=comm_ref.at[j, 0],
                        dst_ref=comm_ref.at[j, k],
                        send_sem=send_sems.at[j, k - 1],
                        recv_sem=recv_sems.at[j, k - 1],
                        device_id=(lax.rem(my + k, N_DEV),),
                        device_id_type=pl.DeviceIdType.MESH,
                    ).wait_send()

    last = N_STEP - 1
    grid = (N_STEP + 1,)
    in_specs = [
            pl.BlockSpec(
                (mb, n_shard), lambda i: (jnp.minimum(i, last), 0),
                memory_space=pltpu.VMEM,
            ),
            pl.BlockSpec(
                (mb, n_shard), lambda i: (jnp.maximum(i, 1) - 1, 0),
                memory_space=pltpu.VMEM,
            ),
            pl.BlockSpec((1, n_shard), lambda i: (0, 0),
                         memory_space=pltpu.VMEM),
            pl.BlockSpec((1, n_shard), lambda i: (0, 0),
                         memory_space=pltpu.VMEM),
    ]
    out_specs = pl.BlockSpec(
        (mb, n_shard), lambda i: (jnp.maximum(i, 1) - 1, 0),
        memory_space=pltpu.VMEM,
    )

    return pl.pallas_call(
        body,
        grid=grid,
        in_specs=in_specs,
        out_specs=out_specs,
        out_shape=jax.ShapeDtypeStruct((m, n_shard), jnp.bfloat16),
        scratch_shapes=[
            pltpu.VMEM((N_STEP, N_DEV, 2, prs, pc), jnp.float32),
            pltpu.SemaphoreType.DMA((N_STEP, N_DEV - 1)),
            pltpu.SemaphoreType.DMA((N_STEP, N_DEV - 1)),
        ],
        compiler_params=(
            pltpu.CompilerParams(dimension_semantics=("arbitrary",))
            if _NO_COMM
            else pltpu.CompilerParams(
                collective_id=0, dimension_semantics=("arbitrary",)
            )
        ),
    )(x, x, gamma2, beta2)


# device time: 14141 ns/iter; 1.2489x vs baseline; 1.2489x over previous
import os

import jax
import jax.numpy as jnp
from jax import lax
from jax.experimental import pallas as pl
from jax.experimental.pallas import tpu as pltpu

N_DEV = 4
EPS = 1e-5
N_PHASE = 4
_NO_COMM = os.environ.get("KERNEL_NO_COMM", "0") == "1"


def kernel(x, gamma, beta):
    m, n_shard = x.shape
    n_global = n_shard * N_DEV
    mh = m // N_PHASE
    pr, pc = mh // 128, 128

    gamma2 = gamma.reshape(1, n_shard)
    beta2 = beta.reshape(1, n_shard)

    def body(x_ref, g_ref, b_ref, out_ref, comm_ref, send_sems, recv_sems):
        my = lax.axis_index("i")

        if not _NO_COMM:
            barrier_sem = pltpu.get_barrier_semaphore()
            for k in range(1, N_DEV):
                pl.semaphore_signal(
                    barrier_sem, inc=1,
                    device_id=(lax.rem(my + k, N_DEV),),
                    device_id_type=pl.DeviceIdType.MESH,
                )

        row = lax.broadcasted_iota(jnp.int32, (mh, pc), 0)
        lane = lax.broadcasted_iota(jnp.int32, (mh, pc), 1)
        mask = (lane == row % pc).astype(jnp.float32)
        sel = (
            lax.broadcasted_iota(jnp.int32, (mh, pr), 1)
            == lax.broadcasted_iota(jnp.int32, (mh, pr), 0) // pc
        ).astype(jnp.float32)

        def pack(s):
            return lax.dot_general(
                sel, s * mask, (((0,), (0,)), ((), ())),
                preferred_element_type=jnp.float32,
            )

        def unpack(t):
            u = lax.dot_general(
                sel, t, (((1,), (0,)), ((), ())),
                preferred_element_type=jnp.float32,
            )
            return jnp.sum(u * mask, axis=1, keepdims=True)

        def partial_sums(p):
            xf = x_ref[pl.ds(p * mh, mh), :]
            s1 = jnp.sum(xf, axis=1, keepdims=True)
            s2 = jnp.sum(xf * xf, axis=1, keepdims=True)
            comm_ref[p, 0, 0] = pack(s1)
            comm_ref[p, 0, 1] = pack(s2)

        def start_sends(p):
            rdmas = []
            for k in range(1, N_DEV):
                rdma = pltpu.make_async_remote_copy(
                    src_ref=comm_ref.at[p, 0],
                    dst_ref=comm_ref.at[p, k],
                    send_sem=send_sems.at[p, k - 1],
                    recv_sem=recv_sems.at[p, k - 1],
                    device_id=(lax.rem(my + k, N_DEV),),
                    device_id_type=pl.DeviceIdType.MESH,
                )
                rdma.start()
                rdmas.append(rdma)
            return rdmas

        gb = g_ref[:, :].astype(jnp.bfloat16)
        bb = b_ref[:, :].astype(jnp.bfloat16)

        def normalize(p, rdmas):
            for rdma in rdmas:
                rdma.wait_recv()
            if _NO_COMM:
                tot1 = comm_ref[p, 0, 0] * 4.0
                tot2 = comm_ref[p, 0, 1] * 4.0
            else:
                tot1 = (comm_ref[p, 0, 0] + comm_ref[p, 1, 0]
                        + comm_ref[p, 2, 0] + comm_ref[p, 3, 0])
                tot2 = (comm_ref[p, 0, 1] + comm_ref[p, 1, 1]
                        + comm_ref[p, 2, 1] + comm_ref[p, 3, 1])
            inv_n = 1.0 / n_global
            mean = unpack(tot1) * inv_n
            var = unpack(tot2) * inv_n - mean * mean
            rstd = lax.rsqrt(var + EPS)
            mean_b = mean.astype(jnp.bfloat16)
            rstd_b = rstd.astype(jnp.bfloat16)
            xb = x_ref[pl.ds(p * mh, mh), :].astype(jnp.bfloat16)
            out_ref[pl.ds(p * mh, mh), :] = (
                (xb - mean_b) * rstd_b * gb + bb
            )

        rdmas = []
        for p in range(N_PHASE):
            partial_sums(p)
            if not _NO_COMM:
                if p == 0:
                    pl.semaphore_wait(barrier_sem, N_DEV - 1)
                rdmas.append(start_sends(p))
            else:
                rdmas.append([])
        for p in range(N_PHASE):
            normalize(p, rdmas[p])
        for group in rdmas:
            for rdma in group:
                rdma.wait_send()

    return pl.pallas_call(
        body,
        out_shape=jax.ShapeDtypeStruct((m, n_shard), jnp.bfloat16),
        in_specs=[
            pl.BlockSpec(memory_space=pltpu.VMEM),
            pl.BlockSpec(memory_space=pltpu.VMEM),
            pl.BlockSpec(memory_space=pltpu.VMEM),
        ],
        out_specs=pl.BlockSpec(memory_space=pltpu.VMEM),
        scratch_shapes=[
            pltpu.VMEM((N_PHASE, N_DEV, 2, pr, pc), jnp.float32),
            pltpu.SemaphoreType.DMA((N_PHASE, N_DEV - 1)),
            pltpu.SemaphoreType.DMA((N_PHASE, N_DEV - 1)),
        ],
        compiler_params=(
            pltpu.CompilerParams()
            if _NO_COMM
            else pltpu.CompilerParams(collective_id=0)
        ),
    )(x, gamma2, beta2)
